# Initial kernel scaffold; baseline (speedup 1.0000x reference)
import jax
import jax.numpy as jnp
from jax.experimental import pallas as pl


def kernel(x, edge_index, bn1_gamma, bn1_beta, bn1_mean, bn1_var, W1, b1, bn2_gamma, bn2_beta, bn2_mean, bn2_var, W2, b2):
    raise NotImplementedError("write your pallas kernel here")


# speedup vs baseline: 10.8297x; 10.8297x over previous
import functools
import math

import jax
import jax.numpy as jnp
from jax import lax
from jax.experimental import pallas as pl
from jax.experimental.pallas import tpu as pltpu
from jax.experimental.pallas import tpu_sc as plsc

N = 10000
E = 320000
D = 128
H = 100
HP = 112
OW = 128
NC = 2
NS = 16
NW = NC * NS
EPW = E // NW
CB = 400
NPAD = 10240
ROWS_PER_TILE = NPAD // NS

_INV_SQRT2 = 1.0 / math.sqrt(2.0)


def _gelu_exact(z):
    return z * 0.5 * (1.0 + lax.erf(z * _INV_SQRT2))


def _prepare_body(x_ref, g_ref, b_ref, m_ref, v_ref, w_ref, bias_ref, y_ref):
    x = x_ref[...]
    xb = (x - m_ref[...]) / jnp.sqrt(v_ref[...] + 1e-3) * g_ref[...] + b_ref[...]
    z = jnp.dot(xb, w_ref[...], preferred_element_type=jnp.float32) + bias_ref[...]
    y = _gelu_exact(z)
    col = lax.broadcasted_iota(jnp.int32, y.shape, 1)
    y_ref[...] = jnp.where(col == HP - 1, 1.0, y)


def _prepare(x, g, b, m, v, w1p, b1p):
    return pl.pallas_call(
        _prepare_body,
        out_shape=jax.ShapeDtypeStruct((N, HP), jnp.float32),
    )(x, g, b, m, v, w1p, b1p)


def _sc_agg_body(y_hbm, src_hbm, dst_hbm, zeros_hbm, out_hbm,
                 src_v, dst_v, rows_v, acc_sh, sem):
    cid = lax.axis_index("c")
    sid = lax.axis_index("s")
    wid = sid * NC + cid

    zbase = sid * ROWS_PER_TILE
    pltpu.sync_copy(zeros_hbm.at[pl.ds(zbase, ROWS_PER_TILE)],
                    acc_sh.at[pl.ds(zbase, ROWS_PER_TILE)])
    plsc.subcore_barrier()

    def step(i, carry):
        base = wid * EPW + i * CB
        pltpu.sync_copy(src_hbm.at[pl.ds(base, CB)], src_v)
        pltpu.async_copy(y_hbm.at[src_v], rows_v, sem).wait()
        pltpu.sync_copy(dst_hbm.at[pl.ds(base, CB)], dst_v)
        pltpu.sync_copy(rows_v, acc_sh.at[dst_v], add=True)
        return carry

    lax.fori_loop(0, EPW // CB, step, 0)
    plsc.subcore_barrier()

    pltpu.sync_copy(acc_sh.at[pl.ds(zbase, ROWS_PER_TILE)],
                    out_hbm.at[pl.ds(cid * NPAD + zbase, ROWS_PER_TILE)])


def _sc_aggregate(y, src, dst, zeros):
    mesh = plsc.VectorSubcoreMesh(core_axis_name="c", subcore_axis_name="s")
    kern = pl.kernel(
        _sc_agg_body,
        out_type=jax.ShapeDtypeStruct((NC * NPAD, HP), jnp.float32),
        mesh=mesh,
        compiler_params=pltpu.CompilerParams(use_tc_tiling_on_sc=False),
        scratch_types=[
            pltpu.VMEM((CB,), jnp.int32),
            pltpu.VMEM((CB,), jnp.int32),
            pltpu.VMEM((CB, HP), jnp.float32),
            pltpu.VMEM_SHARED((NPAD, HP), jnp.float32),
            pltpu.SemaphoreType.DMA,
        ],
    )
    return kern(y, src, dst, zeros)


def _update_body(x_ref, p_ref,
                 gx_ref, bx_ref, mx_ref, vx_ref,
                 ga_ref, ba_ref, ma_ref, va_ref,
                 w2a_ref, w2b_ref, b2_ref, out_ref):
    acc = p_ref[0] + p_ref[1]
    cnt = acc[:, HP - 1:HP]
    mean = jnp.where(cnt > 0.0, acc / jnp.maximum(cnt, 1.0), 0.0)
    hx = (x_ref[...] - mx_ref[...]) / jnp.sqrt(vx_ref[...] + 1e-3) * gx_ref[...] + bx_ref[...]
    ha = (mean - ma_ref[...]) / jnp.sqrt(va_ref[...] + 1e-3) * ga_ref[...] + ba_ref[...]
    z = (jnp.dot(hx, w2a_ref[...], preferred_element_type=jnp.float32)
         + jnp.dot(ha, w2b_ref[...], preferred_element_type=jnp.float32)
         + b2_ref[...])
    out_ref[...] = _gelu_exact(z)


def _update(x, partials, gx, bx, mx, vx, ga, ba, ma, va, w2a, w2b, b2p):
    return pl.pallas_call(
        _update_body,
        out_shape=jax.ShapeDtypeStruct((N, OW), jnp.float32),
    )(x, partials, gx, bx, mx, vx, ga, ba, ma, va, w2a, w2b, b2p)


@jax.jit
def kernel(x, edge_index, bn1_gamma, bn1_beta, bn1_mean, bn1_var, W1, b1,
           bn2_gamma, bn2_beta, bn2_mean, bn2_var, W2, b2):
    f32 = jnp.float32
    row = lambda a: a.reshape(1, -1).astype(f32)

    w1p = jnp.zeros((D, HP), f32).at[:, :H].set(W1)
    b1p = jnp.zeros((1, HP), f32).at[0, :H].set(b1)
    y = _prepare(x, row(bn1_gamma), row(bn1_beta), row(bn1_mean), row(bn1_var),
                 w1p, b1p)

    dst = edge_index[0]
    src = edge_index[1]
    zeros = jnp.zeros((NPAD, HP), f32)
    partials = _sc_aggregate(y, src, dst, zeros).reshape(NC, NPAD, HP)[:, :N, :]

    w2a = jnp.zeros((D, OW), f32).at[:, :H].set(W2[:D])
    w2b = jnp.zeros((HP, OW), f32).at[:H, :H].set(W2[D:])
    b2p = jnp.zeros((1, OW), f32).at[0, :H].set(b2)
    pad = lambda a, fill: jnp.full((1, HP), fill, f32).at[0, :H].set(a[D:])
    ga = pad(bn2_gamma, 0.0)
    ba = pad(bn2_beta, 0.0)
    ma = pad(bn2_mean, 0.0)
    va = pad(bn2_var, 1.0)

    out = _update(x, partials,
                  row(bn2_gamma[:D]), row(bn2_beta[:D]),
                  row(bn2_mean[:D]), row(bn2_var[:D]),
                  ga, ba, ma, va, w2a, w2b, b2p)
    return out[:, :H]


# speedup vs baseline: 13.1879x; 1.2177x over previous
import functools
import math

import jax
import jax.numpy as jnp
from jax import lax
from jax.experimental import pallas as pl
from jax.experimental.pallas import tpu as pltpu
from jax.experimental.pallas import tpu_sc as plsc

N = 10000
E = 320000
D = 128
H = 100
HP = 112
OW = 128
NC = 2
NS = 16
NW = NC * NS
EPW = E // NW
CB = 200
NPAD = 10240
ROWS_PER_TILE = NPAD // NS

_INV_SQRT2 = 1.0 / math.sqrt(2.0)


def _gelu_exact(z):
    return z * 0.5 * (1.0 + lax.erf(z * _INV_SQRT2))


def _prepare_body(x_ref, g_ref, b_ref, m_ref, v_ref, w_ref, bias_ref, y_ref):
    x = x_ref[...]
    xb = (x - m_ref[...]) / jnp.sqrt(v_ref[...] + 1e-3) * g_ref[...] + b_ref[...]
    z = jnp.dot(xb, w_ref[...], preferred_element_type=jnp.float32) + bias_ref[...]
    y = _gelu_exact(z)
    col = lax.broadcasted_iota(jnp.int32, y.shape, 1)
    y_ref[...] = jnp.where(col == HP - 1, 1.0, y)


def _prepare(x, g, b, m, v, w1p, b1p):
    return pl.pallas_call(
        _prepare_body,
        out_shape=jax.ShapeDtypeStruct((N, HP), jnp.float32),
    )(x, g, b, m, v, w1p, b1p)


def _sc_agg_body(y_hbm, src_hbm, dst_hbm, zeros_hbm, out_hbm,
                 src0_v, src1_v, dst0_v, dst1_v, rows0_v, rows1_v, acc_sh,
                 sem0, sem1):
    cid = lax.axis_index("c")
    sid = lax.axis_index("s")
    wid = sid * NC + cid
    ebase = wid * EPW
    nsteps = EPW // CB

    zbase = sid * ROWS_PER_TILE
    pltpu.sync_copy(zeros_hbm.at[pl.ds(zbase, ROWS_PER_TILE)],
                    acc_sh.at[pl.ds(zbase, ROWS_PER_TILE)])
    plsc.subcore_barrier()

    def gather(c, src, rows, sem):
        pltpu.sync_copy(src_hbm.at[pl.ds(ebase + c * CB, CB)], src)
        pltpu.async_copy(y_hbm.at[src], rows, sem)

    def wait(rows, sem):
        pltpu.make_async_copy(y_hbm.at[src0_v], rows, sem).wait()

    pltpu.sync_copy(dst_hbm.at[pl.ds(ebase, CB)], dst0_v)
    gather(0, src0_v, rows0_v, sem0)
    pltpu.sync_copy(dst_hbm.at[pl.ds(ebase + CB, CB)], dst1_v)
    gather(1, src1_v, rows1_v, sem1)

    def step(k, carry):
        wait(rows0_v, sem0)
        pltpu.sync_copy(rows0_v, acc_sh.at[dst0_v], add=True)
        c2 = 2 * k + 2
        pltpu.sync_copy(dst_hbm.at[pl.ds(ebase + c2 * CB, CB)], dst0_v)
        gather(c2, src0_v, rows0_v, sem0)
        wait(rows1_v, sem1)
        pltpu.sync_copy(rows1_v, acc_sh.at[dst1_v], add=True)
        c3 = 2 * k + 3
        pltpu.sync_copy(dst_hbm.at[pl.ds(ebase + c3 * CB, CB)], dst1_v)
        gather(c3, src1_v, rows1_v, sem1)
        return carry

    lax.fori_loop(0, nsteps // 2 - 1, step, 0)
    wait(rows0_v, sem0)
    pltpu.sync_copy(rows0_v, acc_sh.at[dst0_v], add=True)
    wait(rows1_v, sem1)
    pltpu.sync_copy(rows1_v, acc_sh.at[dst1_v], add=True)
    plsc.subcore_barrier()

    pltpu.sync_copy(acc_sh.at[pl.ds(zbase, ROWS_PER_TILE)],
                    out_hbm.at[pl.ds(cid * NPAD + zbase, ROWS_PER_TILE)])


def _sc_aggregate(y, src, dst, zeros):
    mesh = plsc.VectorSubcoreMesh(core_axis_name="c", subcore_axis_name="s")
    kern = pl.kernel(
        _sc_agg_body,
        out_type=jax.ShapeDtypeStruct((NC * NPAD, HP), jnp.float32),
        mesh=mesh,
        compiler_params=pltpu.CompilerParams(use_tc_tiling_on_sc=False),
        scratch_types=[
            pltpu.VMEM((CB,), jnp.int32),
            pltpu.VMEM((CB,), jnp.int32),
            pltpu.VMEM((CB,), jnp.int32),
            pltpu.VMEM((CB,), jnp.int32),
            pltpu.VMEM((CB, HP), jnp.float32),
            pltpu.VMEM((CB, HP), jnp.float32),
            pltpu.VMEM_SHARED((NPAD, HP), jnp.float32),
            pltpu.SemaphoreType.DMA,
            pltpu.SemaphoreType.DMA,
        ],
    )
    return kern(y, src, dst, zeros)


def _update_body(x_ref, p_ref,
                 gx_ref, bx_ref, mx_ref, vx_ref,
                 ga_ref, ba_ref, ma_ref, va_ref,
                 w2a_ref, w2b_ref, b2_ref, out_ref):
    acc = p_ref[pl.ds(0, N)] + p_ref[pl.ds(NPAD, N)]
    cnt = acc[:, HP - 1:HP]
    mean = jnp.where(cnt > 0.0, acc / jnp.maximum(cnt, 1.0), 0.0)
    hx = (x_ref[...] - mx_ref[...]) / jnp.sqrt(vx_ref[...] + 1e-3) * gx_ref[...] + bx_ref[...]
    ha = (mean - ma_ref[...]) / jnp.sqrt(va_ref[...] + 1e-3) * ga_ref[...] + ba_ref[...]
    z = (jnp.dot(hx, w2a_ref[...], preferred_element_type=jnp.float32)
         + jnp.dot(ha, w2b_ref[...], preferred_element_type=jnp.float32)
         + b2_ref[...])
    out_ref[...] = _gelu_exact(z)[:, :H]


def _update(x, partials, gx, bx, mx, vx, ga, ba, ma, va, w2a, w2b, b2p):
    return pl.pallas_call(
        _update_body,
        out_shape=jax.ShapeDtypeStruct((N, H), jnp.float32),
    )(x, partials, gx, bx, mx, vx, ga, ba, ma, va, w2a, w2b, b2p)


@jax.jit
def kernel(x, edge_index, bn1_gamma, bn1_beta, bn1_mean, bn1_var, W1, b1,
           bn2_gamma, bn2_beta, bn2_mean, bn2_var, W2, b2):
    f32 = jnp.float32
    row = lambda a: a.reshape(1, -1).astype(f32)

    w1p = jnp.zeros((D, HP), f32).at[:, :H].set(W1)
    b1p = jnp.zeros((1, HP), f32).at[0, :H].set(b1)
    y = _prepare(x, row(bn1_gamma), row(bn1_beta), row(bn1_mean), row(bn1_var),
                 w1p, b1p)

    dst = edge_index[0]
    src = edge_index[1]
    zeros = jnp.zeros((NPAD, HP), f32)
    partials = _sc_aggregate(y, src, dst, zeros)

    w2a = jnp.zeros((D, OW), f32).at[:, :H].set(W2[:D])
    w2b = jnp.zeros((HP, OW), f32).at[:H, :H].set(W2[D:])
    b2p = jnp.zeros((1, OW), f32).at[0, :H].set(b2)
    pad = lambda a, fill: jnp.full((1, HP), fill, f32).at[0, :H].set(a[D:])
    ga = pad(bn2_gamma, 0.0)
    ba = pad(bn2_beta, 0.0)
    ma = pad(bn2_mean, 0.0)
    va = pad(bn2_var, 1.0)

    return _update(x, partials,
                   row(bn2_gamma[:D]), row(bn2_beta[:D]),
                   row(bn2_mean[:D]), row(bn2_var[:D]),
                   ga, ba, ma, va, w2a, w2b, b2p)


# speedup vs baseline: 15.1989x; 1.1525x over previous
import functools
import math

import jax
import jax.numpy as jnp
from jax import lax
from jax.experimental import pallas as pl
from jax.experimental.pallas import tpu as pltpu
from jax.experimental.pallas import tpu_sc as plsc

N = 10000
E = 320000
D = 128
H = 100
HP = 112
OW = 128
NC = 2
NS = 16
NW = NC * NS
EPW = E // NW
CB = 200
NPAD = 10240
ROWS_PER_TILE = NPAD // NS

_INV_SQRT2 = 1.0 / math.sqrt(2.0)


def _gelu_exact(z):
    return z * 0.5 * (1.0 + lax.erf(z * _INV_SQRT2))


def _prepare_body(x_ref, g_ref, b_ref, m_ref, v_ref, w_ref, bias_ref, y_ref):
    x = x_ref[...]
    xb = (x - m_ref[...]) / jnp.sqrt(v_ref[...] + 1e-3) * g_ref[...] + b_ref[...]
    z = jnp.dot(xb, w_ref[...], preferred_element_type=jnp.float32) + bias_ref[...]
    y = _gelu_exact(z)
    col = lax.broadcasted_iota(jnp.int32, y.shape, 1)
    y_ref[...] = jnp.where(col == HP - 1, 1.0, y)


def _prepare(x, g, b, m, v, w1p, b1p):
    return pl.pallas_call(
        _prepare_body,
        out_shape=jax.ShapeDtypeStruct((N, HP), jnp.float32),
    )(x, g, b, m, v, w1p, b1p)


def _sc_agg_body(y_hbm, edge_hbm, out_hbm,
                 srcA, srcB, srcC, srcD, dstA, dstB, dstC, dstD,
                 rows0_v, rows1_v, acc_sh,
                 sem0, sem1, ssem0, ssem1, is0, is1, is2, is3):
    cid = lax.axis_index("c")
    sid = lax.axis_index("s")
    wid = sid * NC + cid
    ebase = wid * EPW
    nsteps = EPW // CB
    rows = (rows0_v, rows1_v)
    gsem = (sem0, sem1)
    scsem = (ssem0, ssem1)
    isem = (is0, is1, is2, is3)
    srcs = (srcA, srcB, srcC, srcD)
    dsts = (dstA, dstB, dstC, dstD)

    zbase = sid * ROWS_PER_TILE
    zrow = ROWS_PER_TILE // 4
    zval = jnp.zeros((16,), jnp.float32)

    def zstore(i, carry):
        for j in range(HP // 16):
            rows0_v[i, pl.ds(j * 16, 16)] = zval
        return carry

    lax.fori_loop(0, zrow, zstore, 0)
    for q in range(4):
        pltpu.sync_copy(rows0_v.at[pl.ds(0, zrow)],
                        acc_sh.at[pl.ds(zbase + q * zrow, zrow)])
    plsc.subcore_barrier()

    def idx_load_sync(c, j):
        pltpu.sync_copy(edge_hbm.at[1, pl.ds(ebase + c * CB, CB)], srcs[j])
        pltpu.sync_copy(edge_hbm.at[0, pl.ds(ebase + c * CB, CB)], dsts[j])

    def idx_prefetch(c, j):
        pltpu.async_copy(edge_hbm.at[1, pl.ds(ebase + c * CB, CB)],
                         srcs[j], isem[j])
        pltpu.async_copy(edge_hbm.at[0, pl.ds(ebase + c * CB, CB)],
                         dsts[j], isem[j])

    def idx_wait(j):
        for _ in range(2):
            pltpu.make_async_copy(edge_hbm.at[0, pl.ds(0, CB)],
                                  dsts[j], isem[j]).wait()

    def gather(j, b):
        pltpu.async_copy(y_hbm.at[srcs[j]], rows[b], gsem[b])

    def gwait(b):
        pltpu.make_async_copy(y_hbm.at[srcA], rows[b], gsem[b]).wait()

    def ascatter(j, b):
        pltpu.async_copy(rows[b], acc_sh.at[dsts[j]], scsem[b], add=True)

    def swait(b):
        pltpu.make_async_copy(rows[b], acc_sh.at[dstA], scsem[b]).wait()

    idx_load_sync(0, 0)
    idx_load_sync(1, 1)
    gather(0, 0)
    idx_prefetch(2, 2)
    idx_prefetch(3, 3)

    gwait(0)
    ascatter(0, 0)
    gather(1, 1)
    gwait(1)
    ascatter(1, 1)
    swait(0)
    idx_prefetch(4, 0)
    idx_wait(2)
    gather(2, 0)

    def step(k, carry):
        c = 4 * k + 2
        for j in range(4):
            b = j % 2
            gwait(b)
            ascatter((2 + j) % 4, b)
            swait(1 - b)
            idx_prefetch(c + j + 3, (j + 1) % 4)
            idx_wait((j + 3) % 4)
            gather((j + 3) % 4, 1 - b)
        return carry

    lax.fori_loop(0, 11, step, 0)

    gwait(0); ascatter(2, 0); swait(1); idx_prefetch(49, 1)
    idx_wait(3); gather(3, 1)
    gwait(1); ascatter(3, 1); swait(0)
    idx_wait(0); gather(0, 0)
    gwait(0); ascatter(0, 0); swait(1)
    idx_wait(1); gather(1, 1)
    gwait(1); ascatter(1, 1); swait(0); swait(1)
    plsc.subcore_barrier()

    pltpu.sync_copy(acc_sh.at[pl.ds(zbase, ROWS_PER_TILE)],
                    out_hbm.at[pl.ds(cid * NPAD + zbase, ROWS_PER_TILE)])


def _sc_aggregate(y, edge_index):
    mesh = plsc.VectorSubcoreMesh(core_axis_name="c", subcore_axis_name="s")
    kern = pl.kernel(
        _sc_agg_body,
        out_type=jax.ShapeDtypeStruct((NC * NPAD, HP), jnp.float32),
        mesh=mesh,
        compiler_params=pltpu.CompilerParams(use_tc_tiling_on_sc=False),
        scratch_types=[
            pltpu.VMEM((CB,), jnp.int32),
            pltpu.VMEM((CB,), jnp.int32),
            pltpu.VMEM((CB,), jnp.int32),
            pltpu.VMEM((CB,), jnp.int32),
            pltpu.VMEM((CB,), jnp.int32),
            pltpu.VMEM((CB,), jnp.int32),
            pltpu.VMEM((CB,), jnp.int32),
            pltpu.VMEM((CB,), jnp.int32),
            pltpu.VMEM((CB, HP), jnp.float32),
            pltpu.VMEM((CB, HP), jnp.float32),
            pltpu.VMEM_SHARED((NPAD, HP), jnp.float32),
            pltpu.SemaphoreType.DMA,
            pltpu.SemaphoreType.DMA,
            pltpu.SemaphoreType.DMA,
            pltpu.SemaphoreType.DMA,
            pltpu.SemaphoreType.DMA,
            pltpu.SemaphoreType.DMA,
            pltpu.SemaphoreType.DMA,
            pltpu.SemaphoreType.DMA,
        ],
    )
    return kern(y, edge_index)


def _update_body(x_ref, p_ref,
                 gx_ref, bx_ref, mx_ref, vx_ref,
                 ga_ref, ba_ref, ma_ref, va_ref,
                 w2a_ref, w2b_ref, b2_ref, out_ref):
    acc = p_ref[pl.ds(0, N)] + p_ref[pl.ds(NPAD, N)]
    cnt = acc[:, HP - 1:HP]
    mean = jnp.where(cnt > 0.0, acc / jnp.maximum(cnt, 1.0), 0.0)
    hx = (x_ref[...] - mx_ref[...]) / jnp.sqrt(vx_ref[...] + 1e-3) * gx_ref[...] + bx_ref[...]
    ha = (mean - ma_ref[...]) / jnp.sqrt(va_ref[...] + 1e-3) * ga_ref[...] + ba_ref[...]
    z = (jnp.dot(hx, w2a_ref[...], preferred_element_type=jnp.float32)
         + jnp.dot(ha, w2b_ref[...], preferred_element_type=jnp.float32)
         + b2_ref[...])
    out_ref[...] = _gelu_exact(z)[:, :H]


def _update(x, partials, gx, bx, mx, vx, ga, ba, ma, va, w2a, w2b, b2p):
    return pl.pallas_call(
        _update_body,
        out_shape=jax.ShapeDtypeStruct((N, H), jnp.float32),
    )(x, partials, gx, bx, mx, vx, ga, ba, ma, va, w2a, w2b, b2p)


@jax.jit
def kernel(x, edge_index, bn1_gamma, bn1_beta, bn1_mean, bn1_var, W1, b1,
           bn2_gamma, bn2_beta, bn2_mean, bn2_var, W2, b2):
    f32 = jnp.float32
    row = lambda a: a.reshape(1, -1).astype(f32)

    w1p = jnp.zeros((D, HP), f32).at[:, :H].set(W1)
    b1p = jnp.zeros((1, HP), f32).at[0, :H].set(b1)
    y = _prepare(x, row(bn1_gamma), row(bn1_beta), row(bn1_mean), row(bn1_var),
                 w1p, b1p)

    partials = _sc_aggregate(y, edge_index)

    w2a = jnp.zeros((D, OW), f32).at[:, :H].set(W2[:D])
    w2b = jnp.zeros((HP, OW), f32).at[:H, :H].set(W2[D:])
    b2p = jnp.zeros((1, OW), f32).at[0, :H].set(b2)
    pad = lambda a, fill: jnp.full((1, HP), fill, f32).at[0, :H].set(a[D:])
    ga = pad(bn2_gamma, 0.0)
    ba = pad(bn2_beta, 0.0)
    ma = pad(bn2_mean, 0.0)
    va = pad(bn2_var, 1.0)

    return _update(x, partials,
                   row(bn2_gamma[:D]), row(bn2_beta[:D]),
                   row(bn2_mean[:D]), row(bn2_var[:D]),
                   ga, ba, ma, va, w2a, w2b, b2p)


# speedup vs baseline: 16.2959x; 1.0722x over previous
import math

import jax
import jax.numpy as jnp
from jax import lax
from jax.experimental import pallas as pl
from jax.experimental.pallas import tpu as pltpu
from jax.experimental.pallas import tpu_sc as plsc

N = 10000
E = 320000
D = 128
H = 100
HP = 112
OW = 128
NC = 2
NS = 16
NW = NC * NS
EPW = E // NW
CB = 200
NPAD = 10240
ROWS_PER_TILE = NPAD // NS

_INV_SQRT2 = 1.0 / math.sqrt(2.0)


def _gelu_exact(z):
    return z * 0.5 * (1.0 + lax.erf(z * _INV_SQRT2))


def _prepare_body(x_ref, g_ref, b_ref, m_ref, v_ref, w_ref, bias_ref, y_ref):
    x = x_ref[...]
    xb = (x - m_ref[...]) / jnp.sqrt(v_ref[...] + 1e-3) * g_ref[...] + b_ref[...]
    z = jnp.dot(xb, w_ref[...], preferred_element_type=jnp.float32) + bias_ref[...]
    y = _gelu_exact(z)
    col = lax.broadcasted_iota(jnp.int32, y.shape, 1)
    y_ref[...] = jnp.where(col == HP - 1, 1.0, y)


def _prepare(x, g, b, m, v, w1p, b1p):
    return pl.pallas_call(
        _prepare_body,
        out_shape=jax.ShapeDtypeStruct((N, HP), jnp.float32),
    )(x, g, b, m, v, w1p, b1p)


def _sc_agg_body(y_hbm, edge_hbm, out_hbm,
                 srcA, srcB, srcC, srcD, dstA, dstB, dstC, dstD,
                 rows0_v, rows1_v, acc_sh,
                 sem0, sem1, is0, is1, is2, is3):
    cid = lax.axis_index("c")
    sid = lax.axis_index("s")
    wid = sid * NC + cid
    ebase = wid * EPW
    nsteps = EPW // CB
    last = nsteps - 1
    rows = (rows0_v, rows1_v)
    gsem = (sem0, sem1)
    isem = (is0, is1, is2, is3)
    srcs = (srcA, srcB, srcC, srcD)
    dsts = (dstA, dstB, dstC, dstD)

    zbase = sid * ROWS_PER_TILE
    zrow = ROWS_PER_TILE // 4
    zval = jnp.zeros((16,), jnp.float32)

    def zstore(i, carry):
        for j in range(HP // 16):
            rows0_v[i, pl.ds(j * 16, 16)] = zval
        return carry

    lax.fori_loop(0, zrow, zstore, 0)
    for q in range(4):
        pltpu.sync_copy(rows0_v.at[pl.ds(0, zrow)],
                        acc_sh.at[pl.ds(zbase + q * zrow, zrow)])
    plsc.subcore_barrier()

    def idx_load_sync(c, j):
        pltpu.sync_copy(edge_hbm.at[1, pl.ds(ebase + c * CB, CB)], srcs[j])
        pltpu.sync_copy(edge_hbm.at[0, pl.ds(ebase + c * CB, CB)], dsts[j])

    def idx_prefetch(c, j):
        cc = lax.min(c, last)
        pltpu.async_copy(edge_hbm.at[1, pl.ds(ebase + cc * CB, CB)],
                         srcs[j], isem[j])
        pltpu.async_copy(edge_hbm.at[0, pl.ds(ebase + cc * CB, CB)],
                         dsts[j], isem[j])

    def idx_wait(j):
        for _ in range(2):
            pltpu.make_async_copy(edge_hbm.at[0, pl.ds(0, CB)],
                                  dsts[j], isem[j]).wait()

    def gather(j, b):
        pltpu.async_copy(y_hbm.at[srcs[j]], rows[b], gsem[b])

    def gwait(b):
        pltpu.make_async_copy(y_hbm.at[srcA], rows[b], gsem[b]).wait()

    def scatter(j, b):
        pltpu.sync_copy(rows[b], acc_sh.at[dsts[j]], add=True)

    idx_load_sync(0, 0)
    idx_load_sync(1, 1)
    gather(0, 0)
    gather(1, 1)
    idx_prefetch(2, 2)
    idx_prefetch(3, 3)

    def step(k, carry):
        c = 4 * k
        for j in range(4):
            b = j % 2
            gwait(b)
            scatter(j, b)
            idx_prefetch(c + j + 4, j)
            idx_wait((j + 2) % 4)
            gather((j + 2) % 4, b)
        return carry

    lax.fori_loop(0, nsteps // 4, step, 0)
    gwait(0)
    scatter(0, 0)
    gwait(1)
    scatter(1, 1)
    for j in (2, 3):
        idx_wait(j)
    plsc.subcore_barrier()

    pltpu.sync_copy(acc_sh.at[pl.ds(zbase, ROWS_PER_TILE)],
                    out_hbm.at[pl.ds(cid * NPAD + zbase, ROWS_PER_TILE)])


def _sc_aggregate(y, edge_index):
    mesh = plsc.VectorSubcoreMesh(core_axis_name="c", subcore_axis_name="s")
    kern = pl.kernel(
        _sc_agg_body,
        out_type=jax.ShapeDtypeStruct((NC * NPAD, HP), jnp.float32),
        mesh=mesh,
        compiler_params=pltpu.CompilerParams(use_tc_tiling_on_sc=False),
        scratch_types=[
            pltpu.VMEM((CB,), jnp.int32),
            pltpu.VMEM((CB,), jnp.int32),
            pltpu.VMEM((CB,), jnp.int32),
            pltpu.VMEM((CB,), jnp.int32),
            pltpu.VMEM((CB,), jnp.int32),
            pltpu.VMEM((CB,), jnp.int32),
            pltpu.VMEM((CB,), jnp.int32),
            pltpu.VMEM((CB,), jnp.int32),
            pltpu.VMEM((CB, HP), jnp.float32),
            pltpu.VMEM((CB, HP), jnp.float32),
            pltpu.VMEM_SHARED((NPAD, HP), jnp.float32),
            pltpu.SemaphoreType.DMA,
            pltpu.SemaphoreType.DMA,
            pltpu.SemaphoreType.DMA,
            pltpu.SemaphoreType.DMA,
            pltpu.SemaphoreType.DMA,
            pltpu.SemaphoreType.DMA,
        ],
    )
    return kern(y, edge_index)


def _update_body(x_ref, p_ref,
                 gx_ref, bx_ref, mx_ref, vx_ref,
                 ga_ref, ba_ref, ma_ref, va_ref,
                 w2a_ref, w2b_ref, b2_ref, out_ref):
    acc = p_ref[pl.ds(0, N)] + p_ref[pl.ds(NPAD, N)]
    cnt = acc[:, HP - 1:HP]
    mean = jnp.where(cnt > 0.0, acc / jnp.maximum(cnt, 1.0), 0.0)
    hx = (x_ref[...] - mx_ref[...]) / jnp.sqrt(vx_ref[...] + 1e-3) * gx_ref[...] + bx_ref[...]
    ha = (mean - ma_ref[...]) / jnp.sqrt(va_ref[...] + 1e-3) * ga_ref[...] + ba_ref[...]
    z = (jnp.dot(hx, w2a_ref[...], preferred_element_type=jnp.float32)
         + jnp.dot(ha, w2b_ref[...], preferred_element_type=jnp.float32)
         + b2_ref[...])
    out_ref[...] = _gelu_exact(z)[:, :H]


def _update(x, partials, gx, bx, mx, vx, ga, ba, ma, va, w2a, w2b, b2p):
    return pl.pallas_call(
        _update_body,
        out_shape=jax.ShapeDtypeStruct((N, H), jnp.float32),
    )(x, partials, gx, bx, mx, vx, ga, ba, ma, va, w2a, w2b, b2p)


@jax.jit
def kernel(x, edge_index, bn1_gamma, bn1_beta, bn1_mean, bn1_var, W1, b1,
           bn2_gamma, bn2_beta, bn2_mean, bn2_var, W2, b2):
    f32 = jnp.float32
    row = lambda a: a.reshape(1, -1).astype(f32)

    w1p = jnp.zeros((D, HP), f32).at[:, :H].set(W1)
    b1p = jnp.zeros((1, HP), f32).at[0, :H].set(b1)
    y = _prepare(x, row(bn1_gamma), row(bn1_beta), row(bn1_mean), row(bn1_var),
                 w1p, b1p)

    partials = _sc_aggregate(y, edge_index)

    w2a = jnp.zeros((D, OW), f32).at[:, :H].set(W2[:D])
    w2b = jnp.zeros((HP, OW), f32).at[:H, :H].set(W2[D:])
    b2p = jnp.zeros((1, OW), f32).at[0, :H].set(b2)
    pad = lambda a, fill: jnp.full((1, HP), fill, f32).at[0, :H].set(a[D:])
    ga = pad(bn2_gamma, 0.0)
    ba = pad(bn2_beta, 0.0)
    ma = pad(bn2_mean, 0.0)
    va = pad(bn2_var, 1.0)

    return _update(x, partials,
                   row(bn2_gamma[:D]), row(bn2_beta[:D]),
                   row(bn2_mean[:D]), row(bn2_var[:D]),
                   ga, ba, ma, va, w2a, w2b, b2p)
